# baseline (speedup 1.0000x reference)
import functools

import jax
import jax.numpy as jnp
from jax import lax
from jax.experimental import pallas as pl
from jax.experimental.pallas import tpu as pltpu
from jax.experimental.pallas import tpu_sc as plsc

LOG_EPS_ = 1e-05
NC = 2
NS = 16
NW = NC * NS
LANES = 16
CHUNK = 64
GROW = 2 * CHUNK
SUP = 16
D = 128


def _sc_sumsq(x, idx_packed, e_pad):
    n_nodes = x.shape[0]
    epw = e_pad // NW
    nchunks = epw // CHUNK
    nsup = nchunks // SUP
    sup_idx = SUP * GROW

    mesh = plsc.VectorSubcoreMesh(core_axis_name="c", subcore_axis_name="s")

    @functools.partial(
        pl.kernel,
        out_type=jax.ShapeDtypeStruct((e_pad,), jnp.float32),
        mesh=mesh,
        scratch_types=[
            pltpu.VMEM((sup_idx,), jnp.int32),
            pltpu.VMEM((sup_idx,), jnp.int32),
            pltpu.VMEM((GROW, D), jnp.float32),
            pltpu.VMEM((GROW, D), jnp.float32),
            pltpu.VMEM((epw,), jnp.float32),
            pltpu.VMEM((LANES, LANES + 1), jnp.float32),
            pltpu.VMEM_SHARED((n_nodes, D), jnp.float32),
            pltpu.SemaphoreType.DMA,
            pltpu.SemaphoreType.DMA,
            pltpu.SemaphoreType.DMA,
            pltpu.SemaphoreType.DMA,
        ],
        compiler_params=pltpu.CompilerParams(needs_layout_passes=False),
    )
    def k(x_hbm, ip_hbm, out_hbm,
          pidx0, pidx1, rows0, rows1,
          osum, tsc, x_sp, semi0, semi1, semg0, semg1):
        pidx = (pidx0, pidx1)
        rows = (rows0, rows1)
        semi = (semi0, semi1)
        semg = (semg0, semg1)
        sid = lax.axis_index("s")
        cid = lax.axis_index("c")
        wid = sid * NC + cid
        wbase = wid * epw
        ibase = wid * epw * 2

        def idx_start(si, sb):
            off = ibase + si * sup_idx
            pltpu.async_copy(
                ip_hbm.at[pl.ds(off, sup_idx)], pidx[sb], semi[sb])

        def idx_wait(si, sb):
            off = ibase + si * sup_idx
            pltpu.make_async_copy(
                ip_hbm.at[pl.ds(off, sup_idx)], pidx[sb], semi[sb]).wait()

        def g_start(b, sb, sl):
            pltpu.async_copy(
                x_sp.at[pidx[sb].at[pl.ds(sl * GROW, GROW)]],
                rows[b], semg[b])

        def g_wait(b):
            pltpu.make_async_copy(
                x_sp.at[pidx[0].at[pl.ds(0, GROW)]], rows[b],
                semg[b]).wait()

        def compute(ci, b):
            lane_ids = lax.iota(jnp.int32, 16)
            r = rows[b]

            def grp_body(g, _):
                for t in range(0, LANES, 2):
                    e0 = g * LANES + t
                    e1 = e0 + 1
                    a0 = jnp.zeros((LANES,), jnp.float32)
                    a1 = jnp.zeros((LANES,), jnp.float32)
                    for j in range(D // LANES):
                        s0 = r[e0, pl.ds(j * LANES, LANES)]
                        d0 = r[e0 + CHUNK, pl.ds(j * LANES, LANES)]
                        s1 = r[e1, pl.ds(j * LANES, LANES)]
                        d1 = r[e1 + CHUNK, pl.ds(j * LANES, LANES)]
                        f0 = s0 - d0
                        f1 = s1 - d1
                        a0 = a0 + f0 * f0
                        a1 = a1 + f1 * f1
                    tsc[t, pl.ds(0, LANES)] = a0
                    tsc[t + 1, pl.ds(0, LANES)] = a1
                vals = jnp.zeros((LANES,), jnp.float32)
                for c in range(LANES):
                    col = jnp.full((LANES,), c, jnp.int32)
                    vals = vals + plsc.load_gather(tsc, [lane_ids, col])
                osum[pl.ds(ci * CHUNK + g * LANES, LANES)] = vals
                return 0

            lax.fori_loop(0, CHUNK // LANES, grp_body, 0)

        @pl.when(sid == 0)
        def _():
            pltpu.sync_copy(x_hbm, x_sp)

        plsc.subcore_barrier()

        idx_start(0, 0)
        idx_wait(0, 0)
        g_start(0, 0, 0)
        idx_start(1, 1)

        def body2(ci2, _):
            for b in (0, 1):
                ci = ci2 * 2 + b
                si = ci // SUP
                sb = lax.rem(si, 2)
                nb = 1 - b
                nxt = ci + 1
                nsl = lax.rem(nxt, SUP)

                @pl.when(jnp.logical_and(nxt < nchunks, nsl != 0))
                def _():
                    @pl.when(sb == 0)
                    def _():
                        g_start(nb, 0, nsl)

                    @pl.when(sb == 1)
                    def _():
                        g_start(nb, 1, nsl)

                @pl.when(jnp.logical_and(nxt < nchunks, nsl == 0))
                def _():
                    @pl.when(sb == 0)
                    def _():
                        idx_wait(si + 1, 1)
                        g_start(nb, 1, 0)

                    @pl.when(sb == 1)
                    def _():
                        idx_wait(si + 1, 0)
                        g_start(nb, 0, 0)

                g_wait(b)

                @pl.when(jnp.logical_and(
                    lax.rem(ci, SUP) == SUP - 1, si + 2 < nsup))
                def _():
                    @pl.when(sb == 0)
                    def _():
                        idx_start(si + 2, 0)

                    @pl.when(sb == 1)
                    def _():
                        idx_start(si + 2, 1)

                compute(ci, b)
            return 0

        lax.fori_loop(0, nchunks // 2, body2, 0)
        pltpu.sync_copy(osum, out_hbm.at[pl.ds(wbase, epw)])

    return k(x, idx_packed)


def _tc_log(sums):
    e_pad = sums.shape[0]
    s2 = sums.reshape(e_pad // 512, 512)

    def body(s_ref, o_ref):
        o_ref[...] = jnp.log(LOG_EPS_ + jnp.sqrt(s_ref[...]))

    out = pl.pallas_call(
        body,
        out_shape=jax.ShapeDtypeStruct(s2.shape, jnp.float32),
    )(s2)
    return out.reshape(e_pad)


def kernel(x, edge):
    e = edge.shape[1]
    grain = 2 * NW * CHUNK * SUP
    e_pad = ((e + grain - 1) // grain) * grain
    src = jnp.pad(edge[0].astype(jnp.int32), (0, e_pad - e))
    dst = jnp.pad(edge[1].astype(jnp.int32), (0, e_pad - e))
    nchunk_tot = e_pad // CHUNK
    packed = jnp.stack(
        [src.reshape(nchunk_tot, CHUNK), dst.reshape(nchunk_tot, CHUNK)],
        axis=1).reshape(2 * e_pad)
    sums = _sc_sumsq(x, packed, e_pad)
    return _tc_log(sums)[:e]


# speedup vs baseline: 1.1054x; 1.1054x over previous
import functools

import jax
import jax.numpy as jnp
from jax import lax
from jax.experimental import pallas as pl
from jax.experimental.pallas import tpu as pltpu
from jax.experimental.pallas import tpu_sc as plsc

LOG_EPS_ = 1e-05
NC = 2
NS = 16
NW = NC * NS
LANES = 16
CHUNK = 64
SUP = 16
D = 128


def _sc_sumsq(x, src, dst):
    e_pad = src.shape[0]
    n_nodes = x.shape[0]
    epw = e_pad // NW
    nchunks = epw // CHUNK
    nsup = nchunks // SUP
    sup_edges = SUP * CHUNK

    mesh = plsc.VectorSubcoreMesh(core_axis_name="c", subcore_axis_name="s")

    @functools.partial(
        pl.kernel,
        out_type=jax.ShapeDtypeStruct((e_pad,), jnp.float32),
        mesh=mesh,
        scratch_types=[
            pltpu.VMEM((sup_edges,), jnp.int32),
            pltpu.VMEM((sup_edges,), jnp.int32),
            pltpu.VMEM((sup_edges,), jnp.int32),
            pltpu.VMEM((sup_edges,), jnp.int32),
            pltpu.VMEM((CHUNK, D), jnp.float32),
            pltpu.VMEM((CHUNK, D), jnp.float32),
            pltpu.VMEM((CHUNK, D), jnp.float32),
            pltpu.VMEM((CHUNK, D), jnp.float32),
            pltpu.VMEM((epw,), jnp.float32),
            pltpu.VMEM((LANES, LANES + 1), jnp.float32),
            pltpu.VMEM_SHARED((n_nodes, D), jnp.float32),
            pltpu.SemaphoreType.DMA,
            pltpu.SemaphoreType.DMA,
            pltpu.SemaphoreType.DMA,
            pltpu.SemaphoreType.DMA,
        ],
        compiler_params=pltpu.CompilerParams(needs_layout_passes=False),
    )
    def k(x_hbm, s_hbm, d_hbm, out_hbm,
          sidx0, sidx1, didx0, didx1, srows0, srows1, drows0, drows1,
          osum, tsc, x_sp, semi0, semi1, semg0, semg1):
        sidx = (sidx0, sidx1)
        didx = (didx0, didx1)
        srows = (srows0, srows1)
        drows = (drows0, drows1)
        semi = (semi0, semi1)
        semg = (semg0, semg1)
        sid = lax.axis_index("s")
        cid = lax.axis_index("c")
        wbase = (sid * NC + cid) * epw

        def idx_start(si, sb):
            off = wbase + si * sup_edges
            pltpu.async_copy(
                s_hbm.at[pl.ds(off, sup_edges)], sidx[sb], semi[sb])
            pltpu.async_copy(
                d_hbm.at[pl.ds(off, sup_edges)], didx[sb], semi[sb])

        def idx_wait(si, sb):
            off = wbase + si * sup_edges
            pltpu.make_async_copy(
                s_hbm.at[pl.ds(off, sup_edges)], sidx[sb], semi[sb]).wait()
            pltpu.make_async_copy(
                d_hbm.at[pl.ds(off, sup_edges)], didx[sb], semi[sb]).wait()

        def g_start(b, sb, sl):
            soff = sl * CHUNK
            pltpu.async_copy(
                x_sp.at[sidx[sb].at[pl.ds(soff, CHUNK)]], srows[b], semg[b])
            pltpu.async_copy(
                x_sp.at[didx[sb].at[pl.ds(soff, CHUNK)]], drows[b], semg[b])

        def g_wait(b):
            pltpu.make_async_copy(
                x_sp.at[sidx[0].at[pl.ds(0, CHUNK)]], srows[b],
                semg[b]).wait()
            pltpu.make_async_copy(
                x_sp.at[didx[0].at[pl.ds(0, CHUNK)]], drows[b],
                semg[b]).wait()

        def compute(ci, b):
            lane_ids = lax.iota(jnp.int32, 16)
            sr = srows[b]
            dr = drows[b]

            def grp_body(g, _):
                for t in range(0, LANES, 2):
                    e0 = g * LANES + t
                    e1 = e0 + 1
                    a0 = jnp.zeros((LANES,), jnp.float32)
                    a1 = jnp.zeros((LANES,), jnp.float32)
                    for j in range(D // LANES):
                        s0 = sr[e0, pl.ds(j * LANES, LANES)]
                        d0 = dr[e0, pl.ds(j * LANES, LANES)]
                        s1 = sr[e1, pl.ds(j * LANES, LANES)]
                        d1 = dr[e1, pl.ds(j * LANES, LANES)]
                        f0 = s0 - d0
                        f1 = s1 - d1
                        a0 = a0 + f0 * f0
                        a1 = a1 + f1 * f1
                    tsc[t, pl.ds(0, LANES)] = a0
                    tsc[t + 1, pl.ds(0, LANES)] = a1
                vals = jnp.zeros((LANES,), jnp.float32)
                for c in range(LANES):
                    col = jnp.full((LANES,), c, jnp.int32)
                    vals = vals + plsc.load_gather(tsc, [lane_ids, col])
                osum[pl.ds(ci * CHUNK + g * LANES, LANES)] = vals
                return 0

            lax.fori_loop(0, CHUNK // LANES, grp_body, 0)

        idx_start(0, 0)
        idx_start(1, 1)

        rows_per_tile = (n_nodes // NS) // 8 * 8
        pltpu.sync_copy(
            x_hbm.at[pl.ds(sid * rows_per_tile, rows_per_tile)],
            x_sp.at[pl.ds(sid * rows_per_tile, rows_per_tile)])
        tail = n_nodes - NS * rows_per_tile
        if tail:
            @pl.when(sid == 0)
            def _():
                pltpu.sync_copy(
                    x_hbm.at[pl.ds(NS * rows_per_tile, tail)],
                    x_sp.at[pl.ds(NS * rows_per_tile, tail)])

        plsc.subcore_barrier()

        idx_wait(0, 0)
        g_start(0, 0, 0)

        def body2(ci2, _):
            for b in (0, 1):
                ci = ci2 * 2 + b
                si = ci // SUP
                sb = lax.rem(si, 2)
                nb = 1 - b
                nxt = ci + 1
                nsl = lax.rem(nxt, SUP)

                @pl.when(jnp.logical_and(nxt < nchunks, nsl != 0))
                def _():
                    @pl.when(sb == 0)
                    def _():
                        g_start(nb, 0, nsl)

                    @pl.when(sb == 1)
                    def _():
                        g_start(nb, 1, nsl)

                @pl.when(jnp.logical_and(nxt < nchunks, nsl == 0))
                def _():
                    @pl.when(sb == 0)
                    def _():
                        idx_wait(si + 1, 1)
                        g_start(nb, 1, 0)

                    @pl.when(sb == 1)
                    def _():
                        idx_wait(si + 1, 0)
                        g_start(nb, 0, 0)

                g_wait(b)

                @pl.when(jnp.logical_and(
                    lax.rem(ci, SUP) == SUP - 1, si + 2 < nsup))
                def _():
                    @pl.when(sb == 0)
                    def _():
                        idx_start(si + 2, 0)

                    @pl.when(sb == 1)
                    def _():
                        idx_start(si + 2, 1)

                compute(ci, b)
            return 0

        lax.fori_loop(0, nchunks // 2, body2, 0)
        pltpu.sync_copy(osum, out_hbm.at[pl.ds(wbase, epw)])

    return k(x, src, dst)


def _tc_log(sums):
    e_pad = sums.shape[0]
    s2 = sums.reshape(e_pad // 512, 512)

    def body(s_ref, o_ref):
        o_ref[...] = jnp.log(LOG_EPS_ + jnp.sqrt(s_ref[...]))

    out = pl.pallas_call(
        body,
        out_shape=jax.ShapeDtypeStruct(s2.shape, jnp.float32),
    )(s2)
    return out.reshape(e_pad)


def kernel(x, edge):
    e = edge.shape[1]
    grain = 2 * NW * CHUNK * SUP
    e_pad = ((e + grain - 1) // grain) * grain
    src = jnp.pad(edge[0].astype(jnp.int32), (0, e_pad - e))
    dst = jnp.pad(edge[1].astype(jnp.int32), (0, e_pad - e))
    sums = _sc_sumsq(x, src, dst)
    return _tc_log(sums)[:e]


# speedup vs baseline: 1.1987x; 1.0844x over previous
import functools

import jax
import jax.numpy as jnp
from jax import lax
from jax.experimental import pallas as pl
from jax.experimental.pallas import tpu as pltpu
from jax.experimental.pallas import tpu_sc as plsc

LOG_EPS_ = 1e-05
NC = 2
NS = 16
NW = NC * NS
LANES = 16
CHUNK = 64
SUP = 16
D = 128


def _sc_sumsq(x, src, dst):
    e_pad = src.shape[0]
    n_nodes = x.shape[0]
    epw = e_pad // NW
    nchunks = epw // CHUNK
    nsup = nchunks // SUP
    sup_edges = SUP * CHUNK

    mesh = plsc.VectorSubcoreMesh(core_axis_name="c", subcore_axis_name="s")

    @functools.partial(
        pl.kernel,
        out_type=jax.ShapeDtypeStruct((e_pad,), jnp.float32),
        mesh=mesh,
        scratch_types=[
            pltpu.VMEM((sup_edges,), jnp.int32),
            pltpu.VMEM((sup_edges,), jnp.int32),
            pltpu.VMEM((sup_edges,), jnp.int32),
            pltpu.VMEM((sup_edges,), jnp.int32),
            pltpu.VMEM((CHUNK, D), jnp.float32),
            pltpu.VMEM((CHUNK, D), jnp.float32),
            pltpu.VMEM((CHUNK, D), jnp.float32),
            pltpu.VMEM((CHUNK, D), jnp.float32),
            pltpu.VMEM((epw,), jnp.float32),
            pltpu.VMEM((LANES, LANES + 1), jnp.float32),
            pltpu.VMEM_SHARED((n_nodes, D), jnp.float32),
            pltpu.SemaphoreType.DMA,
            pltpu.SemaphoreType.DMA,
            pltpu.SemaphoreType.DMA,
            pltpu.SemaphoreType.DMA,
        ],
        compiler_params=pltpu.CompilerParams(needs_layout_passes=False),
    )
    def k(x_hbm, s_hbm, d_hbm, out_hbm,
          sidx0, sidx1, didx0, didx1, srows0, srows1, drows0, drows1,
          osum, tsc, x_sp, semi0, semi1, semg0, semg1):
        sidx = (sidx0, sidx1)
        didx = (didx0, didx1)
        srows = (srows0, srows1)
        drows = (drows0, drows1)
        semi = (semi0, semi1)
        semg = (semg0, semg1)
        sid = lax.axis_index("s")
        cid = lax.axis_index("c")
        wbase = (sid * NC + cid) * epw

        def idx_start(si, sb):
            off = wbase + si * sup_edges
            pltpu.async_copy(
                s_hbm.at[pl.ds(off, sup_edges)], sidx[sb], semi[sb])
            pltpu.async_copy(
                d_hbm.at[pl.ds(off, sup_edges)], didx[sb], semi[sb])

        def idx_wait(si, sb):
            off = wbase + si * sup_edges
            pltpu.make_async_copy(
                s_hbm.at[pl.ds(off, sup_edges)], sidx[sb], semi[sb]).wait()
            pltpu.make_async_copy(
                d_hbm.at[pl.ds(off, sup_edges)], didx[sb], semi[sb]).wait()

        def g_start(b, sb, sl):
            soff = sl * CHUNK
            pltpu.async_copy(
                x_sp.at[sidx[sb].at[pl.ds(soff, CHUNK)]], srows[b], semg[b])
            pltpu.async_copy(
                x_sp.at[didx[sb].at[pl.ds(soff, CHUNK)]], drows[b], semg[b])

        def g_wait(b):
            pltpu.make_async_copy(
                x_sp.at[sidx[0].at[pl.ds(0, CHUNK)]], srows[b],
                semg[b]).wait()
            pltpu.make_async_copy(
                x_sp.at[didx[0].at[pl.ds(0, CHUNK)]], drows[b],
                semg[b]).wait()

        def compute(ci, b):
            lane_ids = lax.iota(jnp.int32, 16)
            sr = srows[b]
            dr = drows[b]

            def grp_body(g, _):
                for t in range(0, LANES, 8):
                    es = [g * LANES + t + q for q in range(8)]
                    accs = [jnp.zeros((LANES,), jnp.float32)
                            for _ in range(8)]
                    for j in range(D // LANES):
                        for q in range(8):
                            sv = sr[es[q], pl.ds(j * LANES, LANES)]
                            dv = dr[es[q], pl.ds(j * LANES, LANES)]
                            df = sv - dv
                            accs[q] = accs[q] + df * df
                    for q in range(8):
                        tsc[t + q, pl.ds(0, LANES)] = accs[q]
                vals = jnp.zeros((LANES,), jnp.float32)
                for c in range(LANES):
                    col = jnp.full((LANES,), c, jnp.int32)
                    vals = vals + plsc.load_gather(tsc, [lane_ids, col])
                osum[pl.ds(ci * CHUNK + g * LANES, LANES)] = vals
                return 0

            lax.fori_loop(0, CHUNK // LANES, grp_body, 0)

        idx_start(0, 0)
        idx_start(1, 1)

        rows_per_tile = (n_nodes // NS) // 8 * 8
        pltpu.sync_copy(
            x_hbm.at[pl.ds(sid * rows_per_tile, rows_per_tile)],
            x_sp.at[pl.ds(sid * rows_per_tile, rows_per_tile)])
        tail = n_nodes - NS * rows_per_tile
        if tail:
            @pl.when(sid == 0)
            def _():
                pltpu.sync_copy(
                    x_hbm.at[pl.ds(NS * rows_per_tile, tail)],
                    x_sp.at[pl.ds(NS * rows_per_tile, tail)])

        plsc.subcore_barrier()

        idx_wait(0, 0)
        g_start(0, 0, 0)

        def body2(ci2, _):
            for b in (0, 1):
                ci = ci2 * 2 + b
                si = ci // SUP
                sb = lax.rem(si, 2)
                nb = 1 - b
                nxt = ci + 1
                nsl = lax.rem(nxt, SUP)

                @pl.when(jnp.logical_and(nxt < nchunks, nsl != 0))
                def _():
                    @pl.when(sb == 0)
                    def _():
                        g_start(nb, 0, nsl)

                    @pl.when(sb == 1)
                    def _():
                        g_start(nb, 1, nsl)

                @pl.when(jnp.logical_and(nxt < nchunks, nsl == 0))
                def _():
                    @pl.when(sb == 0)
                    def _():
                        idx_wait(si + 1, 1)
                        g_start(nb, 1, 0)

                    @pl.when(sb == 1)
                    def _():
                        idx_wait(si + 1, 0)
                        g_start(nb, 0, 0)

                g_wait(b)

                @pl.when(jnp.logical_and(
                    lax.rem(ci, SUP) == SUP - 1, si + 2 < nsup))
                def _():
                    @pl.when(sb == 0)
                    def _():
                        idx_start(si + 2, 0)

                    @pl.when(sb == 1)
                    def _():
                        idx_start(si + 2, 1)

                compute(ci, b)
            return 0

        lax.fori_loop(0, nchunks // 2, body2, 0)
        pltpu.sync_copy(osum, out_hbm.at[pl.ds(wbase, epw)])

    return k(x, src, dst)


def _tc_log(sums):
    e_pad = sums.shape[0]
    s2 = sums.reshape(e_pad // 512, 512)

    def body(s_ref, o_ref):
        o_ref[...] = jnp.log(LOG_EPS_ + jnp.sqrt(s_ref[...]))

    out = pl.pallas_call(
        body,
        out_shape=jax.ShapeDtypeStruct(s2.shape, jnp.float32),
    )(s2)
    return out.reshape(e_pad)


def kernel(x, edge):
    e = edge.shape[1]
    grain = 2 * NW * CHUNK * SUP
    e_pad = ((e + grain - 1) // grain) * grain
    src = jnp.pad(edge[0].astype(jnp.int32), (0, e_pad - e))
    dst = jnp.pad(edge[1].astype(jnp.int32), (0, e_pad - e))
    sums = _sc_sumsq(x, src, dst)
    return _tc_log(sums)[:e]


# speedup vs baseline: 1.1991x; 1.0004x over previous
import functools

import jax
import jax.numpy as jnp
from jax import lax
from jax.experimental import pallas as pl
from jax.experimental.pallas import tpu as pltpu
from jax.experimental.pallas import tpu_sc as plsc

LOG_EPS_ = 1e-05
NC = 2
NS = 16
NW = NC * NS
LANES = 16
CHUNK = 64
SUP = 16
D = 128


def _sc_sumsq(x, src, dst):
    e_pad = src.shape[0]
    n_nodes = x.shape[0]
    epw = e_pad // NW
    nchunks = epw // CHUNK
    nsup = nchunks // SUP
    sup_edges = SUP * CHUNK

    mesh = plsc.VectorSubcoreMesh(core_axis_name="c", subcore_axis_name="s")

    @functools.partial(
        pl.kernel,
        out_type=jax.ShapeDtypeStruct((e_pad,), jnp.float32),
        mesh=mesh,
        scratch_types=[
            pltpu.VMEM((sup_edges,), jnp.int32),
            pltpu.VMEM((sup_edges,), jnp.int32),
            pltpu.VMEM((sup_edges,), jnp.int32),
            pltpu.VMEM((sup_edges,), jnp.int32),
            pltpu.VMEM((2 * CHUNK, D), jnp.float32),
            pltpu.VMEM((2 * CHUNK, D), jnp.float32),
            pltpu.VMEM((epw,), jnp.float32),
            pltpu.VMEM((LANES, LANES + 1), jnp.float32),
            pltpu.VMEM_SHARED((n_nodes, D), jnp.float32),
            pltpu.SemaphoreType.DMA,
            pltpu.SemaphoreType.DMA,
            pltpu.SemaphoreType.DMA,
            pltpu.SemaphoreType.DMA,
        ],
        compiler_params=pltpu.CompilerParams(needs_layout_passes=False),
    )
    def k(x_hbm, s_hbm, d_hbm, out_hbm,
          sidx0, sidx1, didx0, didx1, rows0, rows1,
          osum, tsc, x_sp, semi0, semi1, semg0, semg1):
        sidx = (sidx0, sidx1)
        didx = (didx0, didx1)
        rows = (rows0, rows1)
        semi = (semi0, semi1)
        semg = (semg0, semg1)
        sid = lax.axis_index("s")
        cid = lax.axis_index("c")
        wbase = (sid * NC + cid) * epw

        def idx_start(si, sb):
            off = wbase + si * sup_edges
            pltpu.async_copy(
                s_hbm.at[pl.ds(off, sup_edges)], sidx[sb], semi[sb])
            pltpu.async_copy(
                d_hbm.at[pl.ds(off, sup_edges)], didx[sb], semi[sb])

        def idx_wait(si, sb):
            off = wbase + si * sup_edges
            pltpu.make_async_copy(
                s_hbm.at[pl.ds(off, sup_edges)], sidx[sb], semi[sb]).wait()
            pltpu.make_async_copy(
                d_hbm.at[pl.ds(off, sup_edges)], didx[sb], semi[sb]).wait()

        def g_start(b, sb, sl):
            soff = sl * CHUNK
            pltpu.async_copy(
                x_sp.at[sidx[sb].at[pl.ds(soff, CHUNK)]],
                rows[b].at[pl.ds(0, CHUNK)], semg[b])
            pltpu.async_copy(
                x_sp.at[didx[sb].at[pl.ds(soff, CHUNK)]],
                rows[b].at[pl.ds(CHUNK, CHUNK)], semg[b])

        def g_wait(b):
            pltpu.make_async_copy(
                x_sp.at[sidx[0].at[pl.ds(0, 2 * CHUNK)]], rows[b],
                semg[b]).wait()

        def compute(ci, b):
            lane_ids = lax.iota(jnp.int32, 16)
            sr = rows[b]
            dr = rows[b]

            def grp_body(g, _):
                for t in range(0, LANES, 8):
                    es = [g * LANES + t + q for q in range(8)]
                    accs = [jnp.zeros((LANES,), jnp.float32)
                            for _ in range(8)]
                    for j in range(D // LANES):
                        for q in range(8):
                            sv = sr[es[q], pl.ds(j * LANES, LANES)]
                            dv = dr[es[q] + CHUNK, pl.ds(j * LANES, LANES)]
                            df = sv - dv
                            accs[q] = accs[q] + df * df
                    for q in range(8):
                        tsc[t + q, pl.ds(0, LANES)] = accs[q]
                vals = jnp.zeros((LANES,), jnp.float32)
                for c in range(LANES):
                    col = jnp.full((LANES,), c, jnp.int32)
                    vals = vals + plsc.load_gather(tsc, [lane_ids, col])
                osum[pl.ds(ci * CHUNK + g * LANES, LANES)] = vals
                return 0

            lax.fori_loop(0, CHUNK // LANES, grp_body, 0)

        idx_start(0, 0)
        idx_start(1, 1)

        rows_per_tile = (n_nodes // NS) // 8 * 8
        pltpu.sync_copy(
            x_hbm.at[pl.ds(sid * rows_per_tile, rows_per_tile)],
            x_sp.at[pl.ds(sid * rows_per_tile, rows_per_tile)])
        tail = n_nodes - NS * rows_per_tile
        if tail:
            @pl.when(sid == 0)
            def _():
                pltpu.sync_copy(
                    x_hbm.at[pl.ds(NS * rows_per_tile, tail)],
                    x_sp.at[pl.ds(NS * rows_per_tile, tail)])

        plsc.subcore_barrier()

        idx_wait(0, 0)
        g_start(0, 0, 0)

        def body2(ci2, _):
            for b in (0, 1):
                ci = ci2 * 2 + b
                si = ci // SUP
                sb = lax.rem(si, 2)
                nb = 1 - b
                nxt = ci + 1
                nsl = lax.rem(nxt, SUP)

                @pl.when(jnp.logical_and(nxt < nchunks, nsl != 0))
                def _():
                    @pl.when(sb == 0)
                    def _():
                        g_start(nb, 0, nsl)

                    @pl.when(sb == 1)
                    def _():
                        g_start(nb, 1, nsl)

                @pl.when(jnp.logical_and(nxt < nchunks, nsl == 0))
                def _():
                    @pl.when(sb == 0)
                    def _():
                        idx_wait(si + 1, 1)
                        g_start(nb, 1, 0)

                    @pl.when(sb == 1)
                    def _():
                        idx_wait(si + 1, 0)
                        g_start(nb, 0, 0)

                g_wait(b)

                @pl.when(jnp.logical_and(
                    lax.rem(ci, SUP) == SUP - 1, si + 2 < nsup))
                def _():
                    @pl.when(sb == 0)
                    def _():
                        idx_start(si + 2, 0)

                    @pl.when(sb == 1)
                    def _():
                        idx_start(si + 2, 1)

                compute(ci, b)
            return 0

        lax.fori_loop(0, nchunks // 2, body2, 0)
        pltpu.sync_copy(osum, out_hbm.at[pl.ds(wbase, epw)])

    return k(x, src, dst)


def _tc_log(sums):
    e_pad = sums.shape[0]
    s2 = sums.reshape(e_pad // 512, 512)

    def body(s_ref, o_ref):
        o_ref[...] = jnp.log(LOG_EPS_ + jnp.sqrt(s_ref[...]))

    out = pl.pallas_call(
        body,
        out_shape=jax.ShapeDtypeStruct(s2.shape, jnp.float32),
    )(s2)
    return out.reshape(e_pad)


def kernel(x, edge):
    e = edge.shape[1]
    grain = 2 * NW * CHUNK * SUP
    e_pad = ((e + grain - 1) // grain) * grain
    src = jnp.pad(edge[0].astype(jnp.int32), (0, e_pad - e))
    dst = jnp.pad(edge[1].astype(jnp.int32), (0, e_pad - e))
    sums = _sc_sumsq(x, src, dst)
    return _tc_log(sums)[:e]
